# baseline (speedup 1.0000x reference)
import functools

import jax
import jax.numpy as jnp
from jax import lax
from jax.experimental import pallas as pl
from jax.experimental.pallas import tpu as pltpu
from jax.experimental.pallas import tpu_sc as plsc

_B = 16384
_L = 16
_V = 1000000
_NC = 2
_NS = 16
_NW = _NC * _NS
_CHUNK = _B // _NW
_IDXW = 128
_NIDX = _CHUNK // _IDXW
_NG = _CHUNK // _L


def _mud_body(users, items, g2, uEmbedF, uBias, itemEmbedF, itemBias, price,
              rmf_uEF, rmf_iEF, rmf_uB, rmf_iB, out,
              idxu, idxi, uT_v, iT_v, ruT_v, riT_v,
              uB_v, iB_v, ruB_v, riB_v, p_v, g_v, out_v, sem):
    wid = lax.axis_index("s") * _NC + lax.axis_index("c")
    base = pl.multiple_of(wid * _CHUNK, _CHUNK)

    for j in range(_NIDX):
        pltpu.sync_copy(users.at[pl.ds(base + j * _IDXW, _IDXW)], idxu.at[j])
        pltpu.sync_copy(items.at[pl.ds(base + j * _IDXW, _IDXW)], idxi.at[j])
    pltpu.sync_copy(g2, g_v)

    descs = []
    for j in range(_NIDX):
        s = pl.ds(j * _IDXW, _IDXW)
        iu = idxu.at[j]
        ii = idxi.at[j]
        for l in range(_L):
            fs = pl.ds(l * _V, _V)
            d = pl.ds(l * _CHUNK + j * _IDXW, _IDXW)
            descs.append(pltpu.async_copy(uEmbedF.at[fs].at[iu], uT_v.at[d], sem))
            descs.append(pltpu.async_copy(rmf_uEF.at[fs].at[iu], ruT_v.at[d], sem))
            descs.append(pltpu.async_copy(itemEmbedF.at[fs].at[ii], iT_v.at[d], sem))
            descs.append(pltpu.async_copy(rmf_iEF.at[fs].at[ii], riT_v.at[d], sem))
        descs.append(pltpu.async_copy(uBias.at[iu], uB_v.at[s], sem))
        descs.append(pltpu.async_copy(rmf_uB.at[iu], ruB_v.at[s], sem))
        descs.append(pltpu.async_copy(itemBias.at[ii], iB_v.at[s], sem))
        descs.append(pltpu.async_copy(rmf_iB.at[ii], riB_v.at[s], sem))
        descs.append(pltpu.async_copy(price.at[ii], p_v.at[s], sem))
    for d in descs:
        d.wait()

    gB = g_v[pl.ds(0, 16)]
    rg = g_v[pl.ds(16, 16)]

    def group(g, carry):
        b = g * _L
        acc_a = jnp.zeros((16,), jnp.float32)
        acc_r = jnp.zeros((16,), jnp.float32)
        for l in range(_L):
            d = pl.ds(l * _CHUNK + b, _L)
            acc_a = acc_a + uT_v[d] * iT_v[d]
            acc_r = acc_r + ruT_v[d] * riT_v[d]
        s = pl.ds(b, _L)
        alpha = gB + uB_v[s] + iB_v[s] + acc_a
        r = rg + ruB_v[s] + riB_v[s] + acc_r
        tanh_r = 1.0 - 2.0 / (jnp.exp(2.0 * r) + 1.0)
        inv_sig = 1.0 + jnp.exp(-p_v[s])
        out_v[s] = 0.5 * alpha * tanh_r * inv_sig
        return carry

    lax.fori_loop(0, _NG, group, 0)
    pltpu.sync_copy(out_v, out.at[pl.ds(base, _CHUNK)])


_mud_sc = functools.partial(
    pl.kernel,
    out_type=jax.ShapeDtypeStruct((_B,), jnp.float32),
    mesh=plsc.VectorSubcoreMesh(core_axis_name="c", subcore_axis_name="s"),
    compiler_params=pltpu.CompilerParams(
        needs_layout_passes=False, use_tc_tiling_on_sc=False),
    scratch_types=[
        pltpu.VMEM((_NIDX, _IDXW), jnp.int32),
        pltpu.VMEM((_NIDX, _IDXW), jnp.int32),
        pltpu.VMEM((_L * _CHUNK,), jnp.float32),
        pltpu.VMEM((_L * _CHUNK,), jnp.float32),
        pltpu.VMEM((_L * _CHUNK,), jnp.float32),
        pltpu.VMEM((_L * _CHUNK,), jnp.float32),
        pltpu.VMEM((_CHUNK,), jnp.float32),
        pltpu.VMEM((_CHUNK,), jnp.float32),
        pltpu.VMEM((_CHUNK,), jnp.float32),
        pltpu.VMEM((_CHUNK,), jnp.float32),
        pltpu.VMEM((_CHUNK,), jnp.float32),
        pltpu.VMEM((32,), jnp.float32),
        pltpu.VMEM((_CHUNK,), jnp.float32),
        pltpu.SemaphoreType.DMA,
    ],
)(_mud_body)


def kernel(users, items, gBias, uBias, itemBias, uEmbed, itemEmbed, price,
           rmf_uE, rmf_iE, rmf_uB, rmf_iB, rmf_g):
    users = users.astype(jnp.int32)
    items = items.astype(jnp.int32)
    g2 = jnp.concatenate([
        jnp.broadcast_to(gBias.reshape(1), (16,)),
        jnp.broadcast_to(rmf_g.reshape(1), (16,)),
    ])
    return _mud_sc(users, items, g2,
                   uEmbed.T.reshape(-1), uBias.reshape(-1),
                   itemEmbed.T.reshape(-1), itemBias.reshape(-1),
                   price, rmf_uE.T.reshape(-1), rmf_iE.T.reshape(-1),
                   rmf_uB.reshape(-1), rmf_iB.reshape(-1))


# speedup vs baseline: 6.7951x; 6.7951x over previous
import functools

import jax
import jax.numpy as jnp
from jax import lax
from jax.experimental import pallas as pl
from jax.experimental.pallas import tpu as pltpu
from jax.experimental.pallas import tpu_sc as plsc

_B = 16384
_L = 16
_V = 1000000
_TH = 999936
_NT = _V - _TH
_NC = 2
_NS = 16
_NW = _NC * _NS
_CHUNK = _B // _NW
_IDXW = 128
_NIDX = _CHUNK // _IDXW
_NG = _CHUNK // _L

_SLAB = 4096
_NFULL = _TH // _SLAB
_KMAX = -(-_NFULL // _NW)


def _detile_body(uE3, iE3, ruE3, riE3, uF, iF, ruF, riF, buf, buf2, sem):
    wid = lax.axis_index("s") * _NC + lax.axis_index("c")

    def _move(src, dst, lh, c, n):
        pltpu.sync_copy(src.at[lh, :, pl.ds(c * _SLAB, n)],
                        buf.at[:, pl.ds(0, n)])

        def tcol(t, carry):
            for ll in range(8):
                for k in range(8):
                    o = t * 128 + k * 16
                    buf2[pl.ds(ll * _SLAB + o, 16)] = buf[ll, pl.ds(o, 16)]
            return carry

        lax.fori_loop(0, n // 128, tcol, 0)
        for ll in range(8):
            l = lh * 8 + ll
            pltpu.sync_copy(buf2.at[pl.ds(ll * _SLAB, n)],
                            dst.at[pl.ds(l * _TH + c * _SLAB, n)])

    units = ((uE3, uF), (iE3, iF), (ruE3, ruF), (riE3, riF))
    for t, (src, dst) in enumerate(units):
        for lh in range(2):
            def body(k, carry, src=src, dst=dst, lh=lh):
                c = wid + k * _NW

                @pl.when(c < _NFULL)
                def _():
                    _move(src, dst, lh, c, _SLAB)

                return carry

            lax.fori_loop(0, _KMAX, body, 0)
            @pl.when(wid == t * 2 + lh)
            def _(src=src, dst=dst, lh=lh):
                _move(src, dst, lh, _NFULL, _TH - _NFULL * _SLAB)


_detile = functools.partial(
    pl.kernel,
    out_type=[jax.ShapeDtypeStruct((_L * _TH,), jnp.float32)] * 4,
    mesh=plsc.VectorSubcoreMesh(core_axis_name="c", subcore_axis_name="s"),
    compiler_params=pltpu.CompilerParams(use_tc_tiling_on_sc=True),
    scratch_types=[
        pltpu.VMEM((8, _SLAB), jnp.float32),
        pltpu.VMEM((8 * _SLAB,), jnp.float32),
        pltpu.SemaphoreType.DMA,
    ],
)(_detile_body)


def _mud_body(users, items, g2, uEmbedF, uBias, itemEmbedF, itemBias, price,
              rmf_uEF, rmf_iEF, rmf_uB, rmf_iB, tail_u, tail_i, out,
              idxu, idxi, idxuc, idxic, uT_v, iT_v, ruT_v, riT_v,
              uB_v, iB_v, ruB_v, riB_v, p_v, g_v, tu_v, ti_v, out_v, sem):
    wid = lax.axis_index("s") * _NC + lax.axis_index("c")
    base = pl.multiple_of(wid * _CHUNK, _CHUNK)

    for j in range(_NIDX):
        pltpu.sync_copy(users.at[pl.ds(base + j * _IDXW, _IDXW)], idxu.at[j])
        pltpu.sync_copy(items.at[pl.ds(base + j * _IDXW, _IDXW)], idxi.at[j])
    pltpu.sync_copy(g2, g_v)
    pltpu.sync_copy(tail_u, tu_v)
    pltpu.sync_copy(tail_i, ti_v)
    for j in range(_NIDX):
        for k in range(_IDXW // 16):
            s16 = pl.ds(k * 16, 16)
            idxuc[j, s16] = jnp.minimum(idxu[j, s16], _TH - 1)
            idxic[j, s16] = jnp.minimum(idxi[j, s16], _TH - 1)

    descs = []
    for j in range(_NIDX):
        s = pl.ds(j * _IDXW, _IDXW)
        iu = idxu.at[j]
        ii = idxi.at[j]
        iuc = idxuc.at[j]
        iic = idxic.at[j]
        for l in range(_L):
            fs = pl.ds(l * _TH, _TH)
            d = pl.ds(l * _CHUNK + j * _IDXW, _IDXW)
            descs.append(pltpu.async_copy(uEmbedF.at[fs].at[iuc], uT_v.at[d], sem))
            descs.append(pltpu.async_copy(rmf_uEF.at[fs].at[iuc], ruT_v.at[d], sem))
            descs.append(pltpu.async_copy(itemEmbedF.at[fs].at[iic], iT_v.at[d], sem))
            descs.append(pltpu.async_copy(rmf_iEF.at[fs].at[iic], riT_v.at[d], sem))
        descs.append(pltpu.async_copy(uBias.at[iu], uB_v.at[s], sem))
        descs.append(pltpu.async_copy(rmf_uB.at[iu], ruB_v.at[s], sem))
        descs.append(pltpu.async_copy(itemBias.at[ii], iB_v.at[s], sem))
        descs.append(pltpu.async_copy(rmf_iB.at[ii], riB_v.at[s], sem))
        descs.append(pltpu.async_copy(price.at[ii], p_v.at[s], sem))
    for d in descs:
        d.wait()

    def fix(sg, carry):
        b0 = sg * _L
        j = sg // 8
        off = (sg % 8) * 16
        ru = idxu[j, pl.ds(off, 16)]
        mu = ru >= _TH
        ri = idxi[j, pl.ds(off, 16)]
        mi = ri >= _TH

        @pl.when(jnp.any(mu))
        def _():
            tl = jnp.maximum(ru - _TH, 0)
            for l in range(_L):
                d = pl.ds(l * _CHUNK + b0, _L)
                tv = plsc.load_gather(tu_v, [l * _NT + tl])
                uT_v[d] = jnp.where(mu, tv, uT_v[d])
                tv2 = plsc.load_gather(tu_v, [_L * _NT + l * _NT + tl])
                ruT_v[d] = jnp.where(mu, tv2, ruT_v[d])

        @pl.when(jnp.any(mi))
        def _():
            tl = jnp.maximum(ri - _TH, 0)
            for l in range(_L):
                d = pl.ds(l * _CHUNK + b0, _L)
                tv = plsc.load_gather(ti_v, [l * _NT + tl])
                iT_v[d] = jnp.where(mi, tv, iT_v[d])
                tv2 = plsc.load_gather(ti_v, [_L * _NT + l * _NT + tl])
                riT_v[d] = jnp.where(mi, tv2, riT_v[d])

        return carry

    lax.fori_loop(0, _NG, fix, 0)

    gB = g_v[pl.ds(0, 16)]
    rg = g_v[pl.ds(16, 16)]

    def group(g, carry):
        b = g * _L
        acc_a = jnp.zeros((16,), jnp.float32)
        acc_r = jnp.zeros((16,), jnp.float32)
        for l in range(_L):
            d = pl.ds(l * _CHUNK + b, _L)
            acc_a = acc_a + uT_v[d] * iT_v[d]
            acc_r = acc_r + ruT_v[d] * riT_v[d]
        s = pl.ds(b, _L)
        alpha = gB + uB_v[s] + iB_v[s] + acc_a
        r = rg + ruB_v[s] + riB_v[s] + acc_r
        tanh_r = 1.0 - 2.0 / (jnp.exp(2.0 * r) + 1.0)
        inv_sig = 1.0 + jnp.exp(-p_v[s])
        out_v[s] = 0.5 * alpha * tanh_r * inv_sig
        return carry

    lax.fori_loop(0, _NG, group, 0)
    pltpu.sync_copy(out_v, out.at[pl.ds(base, _CHUNK)])


_mud_sc = functools.partial(
    pl.kernel,
    out_type=jax.ShapeDtypeStruct((_B,), jnp.float32),
    mesh=plsc.VectorSubcoreMesh(core_axis_name="c", subcore_axis_name="s"),
    compiler_params=pltpu.CompilerParams(
        needs_layout_passes=False, use_tc_tiling_on_sc=False),
    scratch_types=[
        pltpu.VMEM((_NIDX, _IDXW), jnp.int32),
        pltpu.VMEM((_NIDX, _IDXW), jnp.int32),
        pltpu.VMEM((_NIDX, _IDXW), jnp.int32),
        pltpu.VMEM((_NIDX, _IDXW), jnp.int32),
        pltpu.VMEM((_L * _CHUNK,), jnp.float32),
        pltpu.VMEM((_L * _CHUNK,), jnp.float32),
        pltpu.VMEM((_L * _CHUNK,), jnp.float32),
        pltpu.VMEM((_L * _CHUNK,), jnp.float32),
        pltpu.VMEM((_CHUNK,), jnp.float32),
        pltpu.VMEM((_CHUNK,), jnp.float32),
        pltpu.VMEM((_CHUNK,), jnp.float32),
        pltpu.VMEM((_CHUNK,), jnp.float32),
        pltpu.VMEM((_CHUNK,), jnp.float32),
        pltpu.VMEM((32,), jnp.float32),
        pltpu.VMEM((2 * _L * _NT,), jnp.float32),
        pltpu.VMEM((2 * _L * _NT,), jnp.float32),
        pltpu.VMEM((_CHUNK,), jnp.float32),
        pltpu.SemaphoreType.DMA,
    ],
)(_mud_body)


def kernel(users, items, gBias, uBias, itemBias, uEmbed, itemEmbed, price,
           rmf_uE, rmf_iE, rmf_uB, rmf_iB, rmf_g):
    users = users.astype(jnp.int32)
    items = items.astype(jnp.int32)
    g2 = jnp.concatenate([
        jnp.broadcast_to(gBias.reshape(1), (16,)),
        jnp.broadcast_to(rmf_g.reshape(1), (16,)),
    ])
    view = lambda t: t.T.reshape(2, 8, _V)
    uF, iF, ruF, riF = _detile(view(uEmbed), view(itemEmbed),
                               view(rmf_uE), view(rmf_iE))
    tail_u = jnp.concatenate([uEmbed[_TH:].T.reshape(-1),
                              rmf_uE[_TH:].T.reshape(-1)])
    tail_i = jnp.concatenate([itemEmbed[_TH:].T.reshape(-1),
                              rmf_iE[_TH:].T.reshape(-1)])
    return _mud_sc(users, items, g2,
                   uF, uBias.reshape(-1),
                   iF, itemBias.reshape(-1),
                   price, ruF, riF,
                   rmf_uB.reshape(-1), rmf_iB.reshape(-1),
                   tail_u, tail_i)


# speedup vs baseline: 7.4147x; 1.0912x over previous
import functools

import jax
import jax.numpy as jnp
from jax import lax
from jax.experimental import pallas as pl
from jax.experimental.pallas import tpu as pltpu
from jax.experimental.pallas import tpu_sc as plsc

_B = 16384
_L = 16
_V = 1000000
_TH = 999936
_NT = _V - _TH
_NC = 2
_NS = 16
_NW = _NC * _NS
_CHUNK = _B // _NW
_IDXW = 128
_NIDX = _CHUNK // _IDXW
_NG = _CHUNK // _L

_SLAB = 2048
_BLK = 8 * _SLAB
_NFULL = _TH // _SLAB
_NCH = _NFULL + 1
_FLAT = 2 * _NCH * _BLK
_KP = -(-_NFULL // (2 * _NW))


def _detile_body(uE3, iE3, uF, iF,
                 bufA, bufB, buf2A, buf2B, semA, semB):
    wid = lax.axis_index("s") * _NC + lax.axis_index("c")

    def stage(src, dst, lh, c, n, buf, buf2, sem):
        cin = pltpu.async_copy(src.at[lh, :, pl.ds(c * _SLAB, n)],
                               buf.at[:, pl.ds(0, n)], sem)

        def compute_and_store():
            cin.wait()

            def tcol(t, carry):
                for ll in range(8):
                    for k in range(8):
                        o = t * 128 + k * 16
                        buf2[pl.ds(ll * _SLAB + o, 16)] = buf[ll, pl.ds(o, 16)]
                return carry

            lax.fori_loop(0, n // 128, tcol, 0)
            return pltpu.async_copy(
                buf2, dst.at[pl.ds((lh * _NCH + c) * _BLK, _BLK)], sem)

        return compute_and_store

    units = ((uE3, uF), (iE3, iF))
    for t, (src, dst) in enumerate(units):
        for lh in range(2):
            def body(k, carry, src=src, dst=dst, lh=lh):
                c0 = wid + (2 * k) * _NW
                c1 = wid + (2 * k + 1) * _NW

                @pl.when(c0 < _NFULL)
                def _():
                    fin0 = stage(src, dst, lh, c0, _SLAB, bufA, buf2A, semA)

                    @pl.when(c1 < _NFULL)
                    def _():
                        fin1 = stage(src, dst, lh, c1, _SLAB, bufB, buf2B, semB)
                        out0 = fin0()
                        out1 = fin1()
                        out0.wait()
                        out1.wait()

                    @pl.when(c1 >= _NFULL)
                    def _():
                        fin0().wait()

                return carry

            lax.fori_loop(0, _KP, body, 0)
            @pl.when(wid == _NW - 1 - (t * 2 + lh))
            def _(src=src, dst=dst, lh=lh):
                stage(src, dst, lh, _NFULL, _TH - _NFULL * _SLAB,
                      bufA, buf2A, semA)().wait()


_detile = functools.partial(
    pl.kernel,
    out_type=[jax.ShapeDtypeStruct((_FLAT,), jnp.float32)] * 2,
    mesh=plsc.VectorSubcoreMesh(core_axis_name="c", subcore_axis_name="s"),
    compiler_params=pltpu.CompilerParams(use_tc_tiling_on_sc=True),
    scratch_types=[
        pltpu.VMEM((8, _SLAB), jnp.float32),
        pltpu.VMEM((8, _SLAB), jnp.float32),
        pltpu.VMEM((_BLK,), jnp.float32),
        pltpu.VMEM((_BLK,), jnp.float32),
        pltpu.SemaphoreType.DMA,
        pltpu.SemaphoreType.DMA,
    ],
)(_detile_body)


def _mud_body(users, items, g2, uEmbedF, uBias, itemEmbedF, itemBias, price,
              rmf_uEF, rmf_iEF, rmf_uB, rmf_iB, tail_u, tail_i, out,
              idxu, idxi, idxut, idxit, uT_v, iT_v, ruT_v, riT_v,
              uB_v, iB_v, ruB_v, riB_v, p_v, g_v, tu_v, ti_v, out_v, sem):
    wid = lax.axis_index("s") * _NC + lax.axis_index("c")
    base = pl.multiple_of(wid * _CHUNK, _CHUNK)

    for j in range(_NIDX):
        pltpu.sync_copy(users.at[pl.ds(base + j * _IDXW, _IDXW)], idxu.at[j])
        pltpu.sync_copy(items.at[pl.ds(base + j * _IDXW, _IDXW)], idxi.at[j])
    pltpu.sync_copy(g2, g_v)
    pltpu.sync_copy(tail_u, tu_v)
    pltpu.sync_copy(tail_i, ti_v)
    for j in range(_NIDX):
        for k in range(_IDXW // 16):
            s16 = pl.ds(k * 16, 16)
            ru = jnp.minimum(idxu[j, s16], _TH - 1)
            idxut[j, s16] = ((ru >> 11) << 14) + (ru & (_SLAB - 1))
            ri = jnp.minimum(idxi[j, s16], _TH - 1)
            idxit[j, s16] = ((ri >> 11) << 14) + (ri & (_SLAB - 1))

    descs = []
    for j in range(_NIDX):
        s = pl.ds(j * _IDXW, _IDXW)
        iu = idxu.at[j]
        ii = idxi.at[j]
        iut = idxut.at[j]
        iit = idxit.at[j]
        for l in range(_L):
            lh, ll = divmod(l, 8)
            off = lh * _NCH * _BLK + ll * _SLAB
            fs = pl.ds(off, _FLAT - off)
            d = pl.ds(l * _CHUNK + j * _IDXW, _IDXW)
            descs.append(pltpu.async_copy(uEmbedF.at[fs].at[iut], uT_v.at[d], sem))
            descs.append(pltpu.async_copy(rmf_uEF.at[fs].at[iut], ruT_v.at[d], sem))
            descs.append(pltpu.async_copy(itemEmbedF.at[fs].at[iit], iT_v.at[d], sem))
            descs.append(pltpu.async_copy(rmf_iEF.at[fs].at[iit], riT_v.at[d], sem))
        descs.append(pltpu.async_copy(uBias.at[iu], uB_v.at[s], sem))
        descs.append(pltpu.async_copy(rmf_uB.at[iu], ruB_v.at[s], sem))
        descs.append(pltpu.async_copy(itemBias.at[ii], iB_v.at[s], sem))
        descs.append(pltpu.async_copy(rmf_iB.at[ii], riB_v.at[s], sem))
        descs.append(pltpu.async_copy(price.at[ii], p_v.at[s], sem))
    for d in descs:
        d.wait()

    def fix(sg, carry):
        b0 = sg * _L
        j = sg // 8
        off = (sg % 8) * 16
        ru = idxu[j, pl.ds(off, 16)]
        mu = ru >= _TH
        ri = idxi[j, pl.ds(off, 16)]
        mi = ri >= _TH

        @pl.when(jnp.any(mu))
        def _():
            tl = jnp.maximum(ru - _TH, 0)
            for l in range(_L):
                d = pl.ds(l * _CHUNK + b0, _L)
                tv = plsc.load_gather(tu_v, [l * _NT + tl])
                uT_v[d] = jnp.where(mu, tv, uT_v[d])
                tv2 = plsc.load_gather(tu_v, [_L * _NT + l * _NT + tl])
                ruT_v[d] = jnp.where(mu, tv2, ruT_v[d])

        @pl.when(jnp.any(mi))
        def _():
            tl = jnp.maximum(ri - _TH, 0)
            for l in range(_L):
                d = pl.ds(l * _CHUNK + b0, _L)
                tv = plsc.load_gather(ti_v, [l * _NT + tl])
                iT_v[d] = jnp.where(mi, tv, iT_v[d])
                tv2 = plsc.load_gather(ti_v, [_L * _NT + l * _NT + tl])
                riT_v[d] = jnp.where(mi, tv2, riT_v[d])

        return carry

    lax.fori_loop(0, _NG, fix, 0)

    gB = g_v[pl.ds(0, 16)]
    rg = g_v[pl.ds(16, 16)]

    def group(g, carry):
        b = g * _L
        acc_a = jnp.zeros((16,), jnp.float32)
        acc_r = jnp.zeros((16,), jnp.float32)
        for l in range(_L):
            d = pl.ds(l * _CHUNK + b, _L)
            acc_a = acc_a + uT_v[d] * iT_v[d]
            acc_r = acc_r + ruT_v[d] * riT_v[d]
        s = pl.ds(b, _L)
        alpha = gB + uB_v[s] + iB_v[s] + acc_a
        r = rg + ruB_v[s] + riB_v[s] + acc_r
        tanh_r = 1.0 - 2.0 / (jnp.exp(2.0 * r) + 1.0)
        inv_sig = 1.0 + jnp.exp(-p_v[s])
        out_v[s] = 0.5 * alpha * tanh_r * inv_sig
        return carry

    lax.fori_loop(0, _NG, group, 0)
    pltpu.sync_copy(out_v, out.at[pl.ds(base, _CHUNK)])


_mud_sc = functools.partial(
    pl.kernel,
    out_type=jax.ShapeDtypeStruct((_B,), jnp.float32),
    mesh=plsc.VectorSubcoreMesh(core_axis_name="c", subcore_axis_name="s"),
    compiler_params=pltpu.CompilerParams(
        needs_layout_passes=False, use_tc_tiling_on_sc=False),
    scratch_types=[
        pltpu.VMEM((_NIDX, _IDXW), jnp.int32),
        pltpu.VMEM((_NIDX, _IDXW), jnp.int32),
        pltpu.VMEM((_NIDX, _IDXW), jnp.int32),
        pltpu.VMEM((_NIDX, _IDXW), jnp.int32),
        pltpu.VMEM((_L * _CHUNK,), jnp.float32),
        pltpu.VMEM((_L * _CHUNK,), jnp.float32),
        pltpu.VMEM((_L * _CHUNK,), jnp.float32),
        pltpu.VMEM((_L * _CHUNK,), jnp.float32),
        pltpu.VMEM((_CHUNK,), jnp.float32),
        pltpu.VMEM((_CHUNK,), jnp.float32),
        pltpu.VMEM((_CHUNK,), jnp.float32),
        pltpu.VMEM((_CHUNK,), jnp.float32),
        pltpu.VMEM((_CHUNK,), jnp.float32),
        pltpu.VMEM((32,), jnp.float32),
        pltpu.VMEM((2 * _L * _NT,), jnp.float32),
        pltpu.VMEM((2 * _L * _NT,), jnp.float32),
        pltpu.VMEM((_CHUNK,), jnp.float32),
        pltpu.SemaphoreType.DMA,
    ],
)(_mud_body)


def kernel(users, items, gBias, uBias, itemBias, uEmbed, itemEmbed, price,
           rmf_uE, rmf_iE, rmf_uB, rmf_iB, rmf_g):
    users = users.astype(jnp.int32)
    items = items.astype(jnp.int32)
    g2 = jnp.concatenate([
        jnp.broadcast_to(gBias.reshape(1), (16,)),
        jnp.broadcast_to(rmf_g.reshape(1), (16,)),
    ])
    view = lambda t: t.T.reshape(2, 8, _V)
    uF, iF = _detile(view(uEmbed), view(itemEmbed))
    ruF, riF = _detile(view(rmf_uE), view(rmf_iE))
    tail_u = jnp.concatenate([uEmbed[_TH:].T.reshape(-1),
                              rmf_uE[_TH:].T.reshape(-1)])
    tail_i = jnp.concatenate([itemEmbed[_TH:].T.reshape(-1),
                              rmf_iE[_TH:].T.reshape(-1)])
    return _mud_sc(users, items, g2,
                   uF, uBias.reshape(-1),
                   iF, itemBias.reshape(-1),
                   price, ruF, riF,
                   rmf_uB.reshape(-1), rmf_iB.reshape(-1),
                   tail_u, tail_i)


# speedup vs baseline: 8.9671x; 1.2094x over previous
import functools

import jax
import jax.numpy as jnp
from jax import lax
from jax.experimental import pallas as pl
from jax.experimental.pallas import tpu as pltpu
from jax.experimental.pallas import tpu_sc as plsc

_B = 16384
_L = 16
_V = 1000000
_TH = 999936
_NT = _V - _TH
_NC = 2
_NS = 16
_NW = _NC * _NS
_CHUNK = _B // _NW
_IDXW = 128
_NIDX = _CHUNK // _IDXW
_NG = _CHUNK // _L

_SLAB = 2048
_BLK = 8 * _SLAB
_NFULL = _TH // _SLAB
_NCH = _NFULL + 1
_FLAT = 2 * _NCH * _BLK
_KP = -(-_NFULL // (2 * _NW))


def _detile_body(uE3, iE3, uF, iF,
                 bufA, bufB, buf2_0, buf2_1, buf2_2, buf2_3,
                 semA, semB, semO):
    wid = lax.axis_index("s") * _NC + lax.axis_index("c")

    def start_in(src, lh, c, buf, sem):
        return pltpu.async_copy(src.at[lh, :, pl.ds(c * _SLAB, _SLAB)],
                                buf, sem)

    def wait_in(src, lh, c, buf, sem):
        pltpu.make_async_copy(src.at[lh, :, pl.ds(c * _SLAB, _SLAB)],
                              buf, sem).wait()

    def detile(dst, lh, c, n, buf, buf2):
        def tcol(t, carry):
            for ll in range(8):
                for k in range(8):
                    o = t * 128 + k * 16
                    buf2[pl.ds(ll * _SLAB + o, 16)] = buf[ll, pl.ds(o, 16)]
            return carry

        lax.fori_loop(0, n // 128, tcol, 0)
        return pltpu.async_copy(
            buf2, dst.at[pl.ds((lh * _NCH + c) * _BLK, _BLK)], semO)

    units = ((uE3, uF), (iE3, iF))
    for t, (src, dst) in enumerate(units):
        for lh in range(2):
            @pl.when(wid < _NFULL)
            def _(src=src, lh=lh):
                start_in(src, lh, wid, bufA, semA)

            @pl.when(wid + _NW < _NFULL)
            def _(src=src, lh=lh):
                start_in(src, lh, wid + _NW, bufB, semB)

            def body(k, carry, src=src, dst=dst, lh=lh):
                cs = [wid + (4 * k + i) * _NW for i in range(6)]
                bufs = ((bufA, semA, buf2_0), (bufB, semB, buf2_1),
                        (bufA, semA, buf2_2), (bufB, semB, buf2_3))
                for i, (buf, sem, buf2) in enumerate(bufs):
                    @pl.when(cs[i] < _NFULL)
                    def _(i=i, buf=buf, sem=sem, buf2=buf2):
                        wait_in(src, lh, cs[i], buf, sem)
                        detile(dst, lh, cs[i], _SLAB, buf, buf2)

                        @pl.when(cs[i + 2] < _NFULL)
                        def _():
                            start_in(src, lh, cs[i + 2], buf, sem)

                for i, (buf, sem, buf2) in enumerate(bufs):
                    @pl.when(cs[i] < _NFULL)
                    def _(i=i, buf2=buf2):
                        pltpu.make_async_copy(
                            buf2,
                            dst.at[pl.ds((lh * _NCH + cs[i]) * _BLK, _BLK)],
                            semO).wait()
                return carry

            lax.fori_loop(0, -(-_NFULL // (4 * _NW)), body, 0)
            @pl.when(wid == _NW - 1 - (t * 2 + lh))
            def _(src=src, dst=dst, lh=lh):
                n = _TH - _NFULL * _SLAB
                pltpu.sync_copy(src.at[lh, :, pl.ds(_NFULL * _SLAB, n)],
                                bufA.at[:, pl.ds(0, n)])
                detile(dst, lh, _NFULL, n, bufA, buf2_0).wait()


_detile = functools.partial(
    pl.kernel,
    out_type=[jax.ShapeDtypeStruct((_FLAT,), jnp.float32)] * 2,
    mesh=plsc.VectorSubcoreMesh(core_axis_name="c", subcore_axis_name="s"),
    compiler_params=pltpu.CompilerParams(use_tc_tiling_on_sc=True),
    scratch_types=[
        pltpu.VMEM((8, _SLAB), jnp.float32),
        pltpu.VMEM((8, _SLAB), jnp.float32),
        pltpu.VMEM((_BLK,), jnp.float32),
        pltpu.VMEM((_BLK,), jnp.float32),
        pltpu.VMEM((_BLK,), jnp.float32),
        pltpu.VMEM((_BLK,), jnp.float32),
        pltpu.SemaphoreType.DMA,
        pltpu.SemaphoreType.DMA,
        pltpu.SemaphoreType.DMA,
    ],
)(_detile_body)


def _mud_body(users, items, g2, uEmbedF, uBias, itemEmbedF, itemBias, price,
              rmf_uEF, rmf_iEF, rmf_uB, rmf_iB, tail_u, tail_i, out,
              idxu, idxi, idxut, idxit, uT_v, iT_v, ruT_v, riT_v,
              uB_v, iB_v, ruB_v, riB_v, p_v, g_v, tu_v, ti_v, out_v, sem):
    wid = lax.axis_index("s") * _NC + lax.axis_index("c")
    base = pl.multiple_of(wid * _CHUNK, _CHUNK)

    for j in range(_NIDX):
        pltpu.sync_copy(users.at[pl.ds(base + j * _IDXW, _IDXW)], idxu.at[j])
        pltpu.sync_copy(items.at[pl.ds(base + j * _IDXW, _IDXW)], idxi.at[j])
    pltpu.sync_copy(g2, g_v)
    pltpu.sync_copy(tail_u, tu_v)
    pltpu.sync_copy(tail_i, ti_v)
    for j in range(_NIDX):
        for k in range(_IDXW // 16):
            s16 = pl.ds(k * 16, 16)
            ru = jnp.minimum(idxu[j, s16], _TH - 1)
            idxut[j, s16] = ((ru >> 11) << 14) + (ru & (_SLAB - 1))
            ri = jnp.minimum(idxi[j, s16], _TH - 1)
            idxit[j, s16] = ((ri >> 11) << 14) + (ri & (_SLAB - 1))

    descs = []
    for j in range(_NIDX):
        s = pl.ds(j * _IDXW, _IDXW)
        iu = idxu.at[j]
        ii = idxi.at[j]
        iut = idxut.at[j]
        iit = idxit.at[j]
        for l in range(_L):
            lh, ll = divmod(l, 8)
            off = lh * _NCH * _BLK + ll * _SLAB
            fs = pl.ds(off, _FLAT - off)
            d = pl.ds(l * _CHUNK + j * _IDXW, _IDXW)
            descs.append(pltpu.async_copy(uEmbedF.at[fs].at[iut], uT_v.at[d], sem))
            descs.append(pltpu.async_copy(rmf_uEF.at[fs].at[iut], ruT_v.at[d], sem))
            descs.append(pltpu.async_copy(itemEmbedF.at[fs].at[iit], iT_v.at[d], sem))
            descs.append(pltpu.async_copy(rmf_iEF.at[fs].at[iit], riT_v.at[d], sem))
        descs.append(pltpu.async_copy(uBias.at[iu], uB_v.at[s], sem))
        descs.append(pltpu.async_copy(rmf_uB.at[iu], ruB_v.at[s], sem))
        descs.append(pltpu.async_copy(itemBias.at[ii], iB_v.at[s], sem))
        descs.append(pltpu.async_copy(rmf_iB.at[ii], riB_v.at[s], sem))
        descs.append(pltpu.async_copy(price.at[ii], p_v.at[s], sem))
    for d in descs:
        d.wait()

    def fix(sg, carry):
        b0 = sg * _L
        j = sg // 8
        off = (sg % 8) * 16
        ru = idxu[j, pl.ds(off, 16)]
        mu = ru >= _TH
        ri = idxi[j, pl.ds(off, 16)]
        mi = ri >= _TH

        @pl.when(jnp.any(mu))
        def _():
            tl = jnp.maximum(ru - _TH, 0)
            for l in range(_L):
                d = pl.ds(l * _CHUNK + b0, _L)
                tv = plsc.load_gather(tu_v, [l * _NT + tl])
                uT_v[d] = jnp.where(mu, tv, uT_v[d])
                tv2 = plsc.load_gather(tu_v, [_L * _NT + l * _NT + tl])
                ruT_v[d] = jnp.where(mu, tv2, ruT_v[d])

        @pl.when(jnp.any(mi))
        def _():
            tl = jnp.maximum(ri - _TH, 0)
            for l in range(_L):
                d = pl.ds(l * _CHUNK + b0, _L)
                tv = plsc.load_gather(ti_v, [l * _NT + tl])
                iT_v[d] = jnp.where(mi, tv, iT_v[d])
                tv2 = plsc.load_gather(ti_v, [_L * _NT + l * _NT + tl])
                riT_v[d] = jnp.where(mi, tv2, riT_v[d])

        return carry

    lax.fori_loop(0, _NG, fix, 0)

    gB = g_v[pl.ds(0, 16)]
    rg = g_v[pl.ds(16, 16)]

    def group(g, carry):
        b = g * _L
        acc_a = jnp.zeros((16,), jnp.float32)
        acc_r = jnp.zeros((16,), jnp.float32)
        for l in range(_L):
            d = pl.ds(l * _CHUNK + b, _L)
            acc_a = acc_a + uT_v[d] * iT_v[d]
            acc_r = acc_r + ruT_v[d] * riT_v[d]
        s = pl.ds(b, _L)
        alpha = gB + uB_v[s] + iB_v[s] + acc_a
        r = rg + ruB_v[s] + riB_v[s] + acc_r
        tanh_r = 1.0 - 2.0 / (jnp.exp(2.0 * r) + 1.0)
        inv_sig = 1.0 + jnp.exp(-p_v[s])
        out_v[s] = 0.5 * alpha * tanh_r * inv_sig
        return carry

    lax.fori_loop(0, _NG, group, 0)
    pltpu.sync_copy(out_v, out.at[pl.ds(base, _CHUNK)])


_mud_sc = functools.partial(
    pl.kernel,
    out_type=jax.ShapeDtypeStruct((_B,), jnp.float32),
    mesh=plsc.VectorSubcoreMesh(core_axis_name="c", subcore_axis_name="s"),
    compiler_params=pltpu.CompilerParams(
        needs_layout_passes=False, use_tc_tiling_on_sc=False),
    scratch_types=[
        pltpu.VMEM((_NIDX, _IDXW), jnp.int32),
        pltpu.VMEM((_NIDX, _IDXW), jnp.int32),
        pltpu.VMEM((_NIDX, _IDXW), jnp.int32),
        pltpu.VMEM((_NIDX, _IDXW), jnp.int32),
        pltpu.VMEM((_L * _CHUNK,), jnp.float32),
        pltpu.VMEM((_L * _CHUNK,), jnp.float32),
        pltpu.VMEM((_L * _CHUNK,), jnp.float32),
        pltpu.VMEM((_L * _CHUNK,), jnp.float32),
        pltpu.VMEM((_CHUNK,), jnp.float32),
        pltpu.VMEM((_CHUNK,), jnp.float32),
        pltpu.VMEM((_CHUNK,), jnp.float32),
        pltpu.VMEM((_CHUNK,), jnp.float32),
        pltpu.VMEM((_CHUNK,), jnp.float32),
        pltpu.VMEM((32,), jnp.float32),
        pltpu.VMEM((2 * _L * _NT,), jnp.float32),
        pltpu.VMEM((2 * _L * _NT,), jnp.float32),
        pltpu.VMEM((_CHUNK,), jnp.float32),
        pltpu.SemaphoreType.DMA,
    ],
)(_mud_body)


def kernel(users, items, gBias, uBias, itemBias, uEmbed, itemEmbed, price,
           rmf_uE, rmf_iE, rmf_uB, rmf_iB, rmf_g):
    users = users.astype(jnp.int32)
    items = items.astype(jnp.int32)
    g2 = jnp.concatenate([
        jnp.broadcast_to(gBias.reshape(1), (16,)),
        jnp.broadcast_to(rmf_g.reshape(1), (16,)),
    ])
    view = lambda t: t.T.reshape(2, 8, _V)
    uF, iF = _detile(view(uEmbed), view(itemEmbed))
    ruF, riF = _detile(view(rmf_uE), view(rmf_iE))
    tail_u = jnp.concatenate([uEmbed[_TH:].T.reshape(-1),
                              rmf_uE[_TH:].T.reshape(-1)])
    tail_i = jnp.concatenate([itemEmbed[_TH:].T.reshape(-1),
                              rmf_iE[_TH:].T.reshape(-1)])
    return _mud_sc(users, items, g2,
                   uF, uBias.reshape(-1),
                   iF, itemBias.reshape(-1),
                   price, ruF, riF,
                   rmf_uB.reshape(-1), rmf_iB.reshape(-1),
                   tail_u, tail_i)


# speedup vs baseline: 8.9684x; 1.0001x over previous
import functools

import jax
import jax.numpy as jnp
from jax import lax
from jax.experimental import pallas as pl
from jax.experimental.pallas import tpu as pltpu
from jax.experimental.pallas import tpu_sc as plsc

_B = 16384
_L = 16
_V = 1000000
_TH = 999936
_NT = _V - _TH
_NC = 2
_NS = 16
_NW = _NC * _NS
_CHUNK = _B // _NW
_IDXW = 128
_NIDX = _CHUNK // _IDXW
_NG = _CHUNK // _L

_SLAB = 2048
_BLK = 8 * _SLAB
_NFULL = _TH // _SLAB
_NCH = _NFULL + 1
_FLAT = 2 * _NCH * _BLK


def _detile_body(uE3, iE3, uF, iF,
                 bufA, bufB, buf2_0, buf2_1, buf2_2, buf2_3,
                 semA, semB, semO):
    wid = lax.axis_index("s") * _NC + lax.axis_index("c")

    def start_in(src, lh, c, buf, sem):
        return pltpu.async_copy(src.at[lh, :, pl.ds(c * _SLAB, _SLAB)],
                                buf, sem)

    def wait_in(src, lh, c, buf, sem):
        pltpu.make_async_copy(src.at[lh, :, pl.ds(c * _SLAB, _SLAB)],
                              buf, sem).wait()

    def detile(dst, lh, c, n, buf, buf2):
        def tcol(t, carry):
            for ll in range(8):
                for k in range(8):
                    o = t * 128 + k * 16
                    buf2[pl.ds(ll * _SLAB + o, 16)] = buf[ll, pl.ds(o, 16)]
            return carry

        lax.fori_loop(0, n // 128, tcol, 0)
        return pltpu.async_copy(
            buf2, dst.at[pl.ds((lh * _NCH + c) * _BLK, _BLK)], semO)

    units = ((uE3, uF), (iE3, iF))
    for t, (src, dst) in enumerate(units):
        for lh in range(2):
            @pl.when(wid < _NFULL)
            def _(src=src, lh=lh):
                start_in(src, lh, wid, bufA, semA)

            @pl.when(wid + _NW < _NFULL)
            def _(src=src, lh=lh):
                start_in(src, lh, wid + _NW, bufB, semB)

            def body(k, carry, src=src, dst=dst, lh=lh):
                cs = [wid + (4 * k + i) * _NW for i in range(6)]
                bufs = ((bufA, semA, buf2_0), (bufB, semB, buf2_1),
                        (bufA, semA, buf2_2), (bufB, semB, buf2_3))
                for i, (buf, sem, buf2) in enumerate(bufs):
                    @pl.when(cs[i] < _NFULL)
                    def _(i=i, buf=buf, sem=sem, buf2=buf2):
                        wait_in(src, lh, cs[i], buf, sem)
                        detile(dst, lh, cs[i], _SLAB, buf, buf2)

                        @pl.when(cs[i + 2] < _NFULL)
                        def _():
                            start_in(src, lh, cs[i + 2], buf, sem)

                for i, (buf, sem, buf2) in enumerate(bufs):
                    @pl.when(cs[i] < _NFULL)
                    def _(i=i, buf2=buf2):
                        pltpu.make_async_copy(
                            buf2,
                            dst.at[pl.ds((lh * _NCH + cs[i]) * _BLK, _BLK)],
                            semO).wait()
                return carry

            lax.fori_loop(0, -(-_NFULL // (4 * _NW)), body, 0)
            @pl.when(wid == _NW - 1 - (t * 2 + lh))
            def _(src=src, dst=dst, lh=lh):
                n = _TH - _NFULL * _SLAB
                pltpu.sync_copy(src.at[lh, :, pl.ds(_NFULL * _SLAB, n)],
                                bufA.at[:, pl.ds(0, n)])
                detile(dst, lh, _NFULL, n, bufA, buf2_0).wait()


_detile = functools.partial(
    pl.kernel,
    out_type=[jax.ShapeDtypeStruct((_FLAT,), jnp.float32)] * 2,
    mesh=plsc.VectorSubcoreMesh(core_axis_name="c", subcore_axis_name="s"),
    compiler_params=pltpu.CompilerParams(use_tc_tiling_on_sc=True),
    scratch_types=[
        pltpu.VMEM((8, _SLAB), jnp.float32),
        pltpu.VMEM((8, _SLAB), jnp.float32),
        pltpu.VMEM((_BLK,), jnp.float32),
        pltpu.VMEM((_BLK,), jnp.float32),
        pltpu.VMEM((_BLK,), jnp.float32),
        pltpu.VMEM((_BLK,), jnp.float32),
        pltpu.SemaphoreType.DMA,
        pltpu.SemaphoreType.DMA,
        pltpu.SemaphoreType.DMA,
    ],
)(_detile_body)


def _mud_body(users, items, g2, uEmbedF, uBias, itemEmbedF, itemBias, price,
              rmf_uEF, rmf_iEF, rmf_uB, rmf_iB, tail_u, tail_i, out,
              idxu, idxi, idxut, idxit, uT_v, iT_v, ruT_v, riT_v,
              uB_v, iB_v, ruB_v, riB_v, p_v, g_v, tu_v, ti_v, out_v, sem):
    wid = lax.axis_index("s") * _NC + lax.axis_index("c")
    base = pl.multiple_of(wid * _CHUNK, _CHUNK)

    for j in range(_NIDX):
        pltpu.sync_copy(users.at[pl.ds(base + j * _IDXW, _IDXW)], idxu.at[j])
        pltpu.sync_copy(items.at[pl.ds(base + j * _IDXW, _IDXW)], idxi.at[j])
    pltpu.sync_copy(g2, g_v)
    pltpu.sync_copy(tail_u, tu_v)
    pltpu.sync_copy(tail_i, ti_v)
    for j in range(_NIDX):
        for k in range(_IDXW // 16):
            s16 = pl.ds(k * 16, 16)
            ru = jnp.minimum(idxu[j, s16], _TH - 1)
            idxut[j, s16] = ((ru >> 11) << 14) + (ru & (_SLAB - 1))
            ri = jnp.minimum(idxi[j, s16], _TH - 1)
            idxit[j, s16] = ((ri >> 11) << 14) + (ri & (_SLAB - 1))

    descs = []
    for j in range(_NIDX):
        s = pl.ds(j * _IDXW, _IDXW)
        iu = idxu.at[j]
        ii = idxi.at[j]
        iut = idxut.at[j]
        iit = idxit.at[j]
        for l in range(_L):
            lh, ll = divmod(l, 8)
            off = lh * _NCH * _BLK + ll * _SLAB
            fs = pl.ds(off, _FLAT - off)
            d = pl.ds(l * _CHUNK + j * _IDXW, _IDXW)
            descs.append(pltpu.async_copy(uEmbedF.at[fs].at[iut], uT_v.at[d], sem))
            descs.append(pltpu.async_copy(rmf_uEF.at[fs].at[iut], ruT_v.at[d], sem))
            descs.append(pltpu.async_copy(itemEmbedF.at[fs].at[iit], iT_v.at[d], sem))
            descs.append(pltpu.async_copy(rmf_iEF.at[fs].at[iit], riT_v.at[d], sem))
        descs.append(pltpu.async_copy(uBias.at[iu], uB_v.at[s], sem))
        descs.append(pltpu.async_copy(rmf_uB.at[iu], ruB_v.at[s], sem))
        descs.append(pltpu.async_copy(itemBias.at[ii], iB_v.at[s], sem))
        descs.append(pltpu.async_copy(rmf_iB.at[ii], riB_v.at[s], sem))
        descs.append(pltpu.async_copy(price.at[ii], p_v.at[s], sem))
    for d in descs:
        d.wait()

    def fix(sg, carry):
        b0 = sg * _L
        j = sg // 8
        off = (sg % 8) * 16
        ru = idxu[j, pl.ds(off, 16)]
        mu = ru >= _TH
        ri = idxi[j, pl.ds(off, 16)]
        mi = ri >= _TH

        @pl.when(jnp.any(mu))
        def _():
            tl = jnp.maximum(ru - _TH, 0)
            for l in range(_L):
                d = pl.ds(l * _CHUNK + b0, _L)
                tv = plsc.load_gather(tu_v, [l * _NT + tl])
                uT_v[d] = jnp.where(mu, tv, uT_v[d])
                tv2 = plsc.load_gather(tu_v, [_L * _NT + l * _NT + tl])
                ruT_v[d] = jnp.where(mu, tv2, ruT_v[d])

        @pl.when(jnp.any(mi))
        def _():
            tl = jnp.maximum(ri - _TH, 0)
            for l in range(_L):
                d = pl.ds(l * _CHUNK + b0, _L)
                tv = plsc.load_gather(ti_v, [l * _NT + tl])
                iT_v[d] = jnp.where(mi, tv, iT_v[d])
                tv2 = plsc.load_gather(ti_v, [_L * _NT + l * _NT + tl])
                riT_v[d] = jnp.where(mi, tv2, riT_v[d])

        return carry

    lax.fori_loop(0, _NG, fix, 0)

    gB = g_v[pl.ds(0, 16)]
    rg = g_v[pl.ds(16, 16)]

    def group(g, carry):
        b = g * _L
        acc_a = jnp.zeros((16,), jnp.float32)
        acc_r = jnp.zeros((16,), jnp.float32)
        for l in range(_L):
            d = pl.ds(l * _CHUNK + b, _L)
            acc_a = acc_a + uT_v[d] * iT_v[d]
            acc_r = acc_r + ruT_v[d] * riT_v[d]
        s = pl.ds(b, _L)
        alpha = gB + uB_v[s] + iB_v[s] + acc_a
        r = rg + ruB_v[s] + riB_v[s] + acc_r
        tanh_r = 1.0 - 2.0 / (jnp.exp(2.0 * r) + 1.0)
        inv_sig = 1.0 + jnp.exp(-p_v[s])
        out_v[s] = 0.5 * alpha * tanh_r * inv_sig
        return carry

    lax.fori_loop(0, _NG, group, 0)
    pltpu.sync_copy(out_v, out.at[pl.ds(base, _CHUNK)])


_mud_sc = functools.partial(
    pl.kernel,
    out_type=jax.ShapeDtypeStruct((_B,), jnp.float32),
    mesh=plsc.VectorSubcoreMesh(core_axis_name="c", subcore_axis_name="s"),
    compiler_params=pltpu.CompilerParams(
        needs_layout_passes=False, use_tc_tiling_on_sc=False),
    scratch_types=[
        pltpu.VMEM((_NIDX, _IDXW), jnp.int32),
        pltpu.VMEM((_NIDX, _IDXW), jnp.int32),
        pltpu.VMEM((_NIDX, _IDXW), jnp.int32),
        pltpu.VMEM((_NIDX, _IDXW), jnp.int32),
        pltpu.VMEM((_L * _CHUNK,), jnp.float32),
        pltpu.VMEM((_L * _CHUNK,), jnp.float32),
        pltpu.VMEM((_L * _CHUNK,), jnp.float32),
        pltpu.VMEM((_L * _CHUNK,), jnp.float32),
        pltpu.VMEM((_CHUNK,), jnp.float32),
        pltpu.VMEM((_CHUNK,), jnp.float32),
        pltpu.VMEM((_CHUNK,), jnp.float32),
        pltpu.VMEM((_CHUNK,), jnp.float32),
        pltpu.VMEM((_CHUNK,), jnp.float32),
        pltpu.VMEM((32,), jnp.float32),
        pltpu.VMEM((2 * _L * _NT,), jnp.float32),
        pltpu.VMEM((2 * _L * _NT,), jnp.float32),
        pltpu.VMEM((_CHUNK,), jnp.float32),
        pltpu.SemaphoreType.DMA,
    ],
)(_mud_body)


def kernel(users, items, gBias, uBias, itemBias, uEmbed, itemEmbed, price,
           rmf_uE, rmf_iE, rmf_uB, rmf_iB, rmf_g):
    users = users.astype(jnp.int32)
    items = items.astype(jnp.int32)
    g2 = jnp.concatenate([
        jnp.broadcast_to(gBias.reshape(1), (16,)),
        jnp.broadcast_to(rmf_g.reshape(1), (16,)),
    ])
    view = lambda t: t.T.reshape(2, 8, _V)
    uF, iF = _detile(view(uEmbed), view(itemEmbed))
    ruF, riF = _detile(view(rmf_uE), view(rmf_iE))
    tail_u = jnp.concatenate([uEmbed[_TH:].T.reshape(-1),
                              rmf_uE[_TH:].T.reshape(-1)])
    tail_i = jnp.concatenate([itemEmbed[_TH:].T.reshape(-1),
                              rmf_iE[_TH:].T.reshape(-1)])
    return _mud_sc(users, items, g2,
                   uF, uBias.reshape(-1),
                   iF, itemBias.reshape(-1),
                   price, ruF, riF,
                   rmf_uB.reshape(-1), rmf_iB.reshape(-1),
                   tail_u, tail_i)
